# Initial kernel scaffold; baseline (speedup 1.0000x reference)
import jax
import jax.numpy as jnp
from jax.experimental import pallas as pl


def kernel(vertex_feat, neighbors_idx, valid_lens, W_embed, Ws, Bs, Wc1, bc1, Wc2, bc2):
    raise NotImplementedError("write your pallas kernel here")


# speedup vs baseline: 11.7651x; 11.7651x over previous
import functools

import jax
import jax.numpy as jnp
from jax import lax
from jax.experimental import pallas as pl
from jax.experimental.pallas import tpu as pltpu
from jax.experimental.pallas import tpu_sc as plsc

_B, _N, _K, _D, _H, _C = 2, 10000, 16, 128, 128, 64
_M = _B * _N
_NW = 32
_MP = 20480
_NPW = _MP // _NW
_CH = 128
_NCH = _NPW // _CH
_BM = 1024


def _sc_aggregate(h, idx4):
    mesh = plsc.VectorSubcoreMesh(core_axis_name="c", subcore_axis_name="s")

    @functools.partial(
        pl.kernel,
        out_type=jax.ShapeDtypeStruct((_MP, _H), jnp.float32),
        mesh=mesh,
        scratch_types=[
            pltpu.VMEM((_K, _NCH, _CH), jnp.int32),
            pltpu.VMEM((_NPW, _H), jnp.float32),
            pltpu.SemaphoreType.DMA,
        ],
    )
    def agg_kernel(h_hbm, idx_hbm, out_hbm, idx_v, acc_v, sem):
        wid = lax.axis_index("s") * 2 + lax.axis_index("c")
        base = wid * _NPW
        pltpu.sync_copy(idx_hbm.at[:, wid], idx_v)

        descs = []
        for ch in range(_NCH):
            descs.append(pltpu.async_copy(
                h_hbm.at[idx_v.at[0, ch]],
                acc_v.at[pl.ds(ch * _CH, _CH), :],
                sem))
        for d in descs:
            d.wait()

        def round_k(kk, carry):
            descs = []
            for ch in range(_NCH):
                descs.append(pltpu.async_copy(
                    h_hbm.at[idx_v.at[kk, ch]],
                    acc_v.at[pl.ds(ch * _CH, _CH), :],
                    sem, add=True))
            for d in descs:
                d.wait()
            return carry

        lax.fori_loop(1, _K, round_k, 0)
        pltpu.sync_copy(acc_v, out_hbm.at[pl.ds(base, _NPW)])

    return agg_kernel(h, idx4)


def _embed_body(x_ref, w_ref, o_ref):
    x = x_ref[...]
    s = jnp.sum(x, axis=1, keepdims=True)
    o_ref[...] = jnp.dot(x, w_ref[...],
                         preferred_element_type=jnp.float32) / s


def _tc_embed(x, w):
    grid = (_MP // _BM,)
    return pl.pallas_call(
        _embed_body,
        grid=grid,
        in_specs=[
            pl.BlockSpec((_BM, _D), lambda i: (i, 0)),
            pl.BlockSpec((_D, _H), lambda i: (0, 0)),
        ],
        out_specs=pl.BlockSpec((_BM, _H), lambda i: (i, 0)),
        out_shape=jax.ShapeDtypeStruct((_MP, _H), jnp.float32),
    )(x, w)


def _layer_body(agg_ref, h_ref, ivl_ref, ws_ref, bs_ref, o_ref):
    a = agg_ref[...] * ivl_ref[...]
    o_ref[...] = jnp.maximum(
        jnp.dot(a, ws_ref[...], preferred_element_type=jnp.float32)
        + jnp.dot(h_ref[...], bs_ref[...],
                  preferred_element_type=jnp.float32),
        0.0)


def _tc_layer(agg, h, ivl, ws, bs):
    grid = (_MP // _BM,)
    return pl.pallas_call(
        _layer_body,
        grid=grid,
        in_specs=[
            pl.BlockSpec((_BM, _H), lambda i: (i, 0)),
            pl.BlockSpec((_BM, _H), lambda i: (i, 0)),
            pl.BlockSpec((_BM, 1), lambda i: (i, 0)),
            pl.BlockSpec((_H, _H), lambda i: (0, 0)),
            pl.BlockSpec((_H, _H), lambda i: (0, 0)),
        ],
        out_specs=pl.BlockSpec((_BM, _H), lambda i: (i, 0)),
        out_shape=jax.ShapeDtypeStruct((_MP, _H), jnp.float32),
    )(agg, h, ivl, ws, bs)


def _head_body(h_ref, w1_ref, b1_ref, w2_ref, b2_ref, o_ref):
    z = jnp.maximum(
        jnp.dot(h_ref[...], w1_ref[...],
                preferred_element_type=jnp.float32) + b1_ref[...],
        0.0)
    logits = jnp.dot(z, w2_ref[...],
                     preferred_element_type=jnp.float32) + b2_ref[...]
    m = jnp.max(logits, axis=1, keepdims=True)
    e = jnp.exp(logits - m)
    o_ref[...] = e / jnp.sum(e, axis=1, keepdims=True)


def _tc_head(h, w1, b1, w2, b2):
    grid = (_MP // _BM,)
    return pl.pallas_call(
        _head_body,
        grid=grid,
        in_specs=[
            pl.BlockSpec((_BM, _H), lambda i: (i, 0)),
            pl.BlockSpec((_H, _H), lambda i: (0, 0)),
            pl.BlockSpec((1, _H), lambda i: (0, 0)),
            pl.BlockSpec((_H, _C), lambda i: (0, 0)),
            pl.BlockSpec((1, _C), lambda i: (0, 0)),
        ],
        out_specs=pl.BlockSpec((_BM, _C), lambda i: (i, 0)),
        out_shape=jax.ShapeDtypeStruct((_MP, _C), jnp.float32),
    )(h, w1, b1, w2, b2)


def kernel(vertex_feat, neighbors_idx, valid_lens, W_embed, Ws, Bs,
           Wc1, bc1, Wc2, bc2):
    x = vertex_feat.reshape(_M, _D)
    xp = jnp.pad(x, ((0, _MP - _M), (0, 0)), constant_values=1.0)

    offs = (jnp.arange(_B, dtype=jnp.int32) * _N)[:, None, None]
    idx = (neighbors_idx + offs).reshape(_M, _K)
    idx = jnp.pad(idx, ((0, _MP - _M), (0, 0)))
    idx4 = idx.T.reshape(_K, _NW, _NCH, _CH)

    vl = jnp.maximum(valid_lens, 1).astype(jnp.float32).reshape(_M, 1)
    ivl = jnp.pad(1.0 / vl, ((0, _MP - _M), (0, 0)), constant_values=1.0)

    h = _tc_embed(xp, W_embed)
    for l in range(Ws.shape[0]):
        agg = _sc_aggregate(h, idx4)
        h = _tc_layer(agg, h, ivl, Ws[l], Bs[l])
    probs = _tc_head(h, Wc1, bc1.reshape(1, _H), Wc2, bc2.reshape(1, _C))
    return probs[:_M].reshape(_B, _N, _C)


# speedup vs baseline: 12.0357x; 1.0230x over previous
import functools

import jax
import jax.numpy as jnp
from jax import lax
from jax.experimental import pallas as pl
from jax.experimental.pallas import tpu as pltpu
from jax.experimental.pallas import tpu_sc as plsc

_B, _N, _K, _D, _H, _C = 2, 10000, 16, 128, 128, 64
_M = _B * _N
_NW = 32
_MP = 20480
_NPW = _MP // _NW
_CH = 128
_NCH = _NPW // _CH
_BM = 1024


def _sc_aggregate(h, idx4):
    mesh = plsc.VectorSubcoreMesh(core_axis_name="c", subcore_axis_name="s")

    @functools.partial(
        pl.kernel,
        out_type=jax.ShapeDtypeStruct((_MP, _H), jnp.float32),
        mesh=mesh,
        scratch_types=[
            pltpu.VMEM((_K, _NCH, _CH), jnp.int32),
            pltpu.VMEM((_NPW, _H), jnp.float32),
            pltpu.SemaphoreType.DMA,
        ],
    )
    def agg_kernel(h_hbm, idx_hbm, out_hbm, idx_v, acc_v, sem):
        wid = lax.axis_index("s") * 2 + lax.axis_index("c")
        base = wid * _NPW
        pltpu.sync_copy(idx_hbm.at[:, wid], idx_v)

        descs = []
        for ch in range(_NCH):
            descs.append(pltpu.async_copy(
                h_hbm.at[idx_v.at[0, ch]],
                acc_v.at[pl.ds(ch * _CH, _CH), :],
                sem))
        for d in descs:
            d.wait()

        def fire_k(kk, carry):
            for ch in range(_NCH):
                pltpu.async_copy(
                    h_hbm.at[idx_v.at[kk, ch]],
                    acc_v.at[pl.ds(ch * _CH, _CH), :],
                    sem, add=True)
            return carry

        lax.fori_loop(1, _K, fire_k, 0)

        def drain_k(kk, carry):
            for ch in range(_NCH):
                pltpu.make_async_copy(
                    h_hbm.at[idx_v.at[kk, ch]],
                    acc_v.at[pl.ds(ch * _CH, _CH), :],
                    sem).wait()
            return carry

        lax.fori_loop(1, _K, drain_k, 0)
        pltpu.sync_copy(acc_v, out_hbm.at[pl.ds(base, _NPW)])

    return agg_kernel(h, idx4)


def _embed_body(x_ref, w_ref, o_ref):
    x = x_ref[...]
    s = jnp.sum(x, axis=1, keepdims=True)
    o_ref[...] = jnp.dot(x, w_ref[...],
                         preferred_element_type=jnp.float32) / s


def _tc_embed(x, w):
    grid = (_MP // _BM,)
    return pl.pallas_call(
        _embed_body,
        grid=grid,
        in_specs=[
            pl.BlockSpec((_BM, _D), lambda i: (i, 0)),
            pl.BlockSpec((_D, _H), lambda i: (0, 0)),
        ],
        out_specs=pl.BlockSpec((_BM, _H), lambda i: (i, 0)),
        out_shape=jax.ShapeDtypeStruct((_MP, _H), jnp.float32),
    )(x, w)


def _layer_body(agg_ref, h_ref, ivl_ref, ws_ref, bs_ref, o_ref):
    a = agg_ref[...] * ivl_ref[...]
    o_ref[...] = jnp.maximum(
        jnp.dot(a, ws_ref[...], preferred_element_type=jnp.float32)
        + jnp.dot(h_ref[...], bs_ref[...],
                  preferred_element_type=jnp.float32),
        0.0)


def _tc_layer(agg, h, ivl, ws, bs):
    grid = (_MP // _BM,)
    return pl.pallas_call(
        _layer_body,
        grid=grid,
        in_specs=[
            pl.BlockSpec((_BM, _H), lambda i: (i, 0)),
            pl.BlockSpec((_BM, _H), lambda i: (i, 0)),
            pl.BlockSpec((_BM, 1), lambda i: (i, 0)),
            pl.BlockSpec((_H, _H), lambda i: (0, 0)),
            pl.BlockSpec((_H, _H), lambda i: (0, 0)),
        ],
        out_specs=pl.BlockSpec((_BM, _H), lambda i: (i, 0)),
        out_shape=jax.ShapeDtypeStruct((_MP, _H), jnp.float32),
    )(agg, h, ivl, ws, bs)


def _head_body(h_ref, w1_ref, b1_ref, w2_ref, b2_ref, o_ref):
    z = jnp.maximum(
        jnp.dot(h_ref[...], w1_ref[...],
                preferred_element_type=jnp.float32) + b1_ref[...],
        0.0)
    logits = jnp.dot(z, w2_ref[...],
                     preferred_element_type=jnp.float32) + b2_ref[...]
    m = jnp.max(logits, axis=1, keepdims=True)
    e = jnp.exp(logits - m)
    o_ref[...] = e / jnp.sum(e, axis=1, keepdims=True)


def _tc_head(h, w1, b1, w2, b2):
    grid = (_MP // _BM,)
    return pl.pallas_call(
        _head_body,
        grid=grid,
        in_specs=[
            pl.BlockSpec((_BM, _H), lambda i: (i, 0)),
            pl.BlockSpec((_H, _H), lambda i: (0, 0)),
            pl.BlockSpec((1, _H), lambda i: (0, 0)),
            pl.BlockSpec((_H, _C), lambda i: (0, 0)),
            pl.BlockSpec((1, _C), lambda i: (0, 0)),
        ],
        out_specs=pl.BlockSpec((_BM, _C), lambda i: (i, 0)),
        out_shape=jax.ShapeDtypeStruct((_MP, _C), jnp.float32),
    )(h, w1, b1, w2, b2)


def kernel(vertex_feat, neighbors_idx, valid_lens, W_embed, Ws, Bs,
           Wc1, bc1, Wc2, bc2):
    x = vertex_feat.reshape(_M, _D)
    xp = jnp.pad(x, ((0, _MP - _M), (0, 0)), constant_values=1.0)

    offs = (jnp.arange(_B, dtype=jnp.int32) * _N)[:, None, None]
    idx = (neighbors_idx + offs).reshape(_M, _K)
    idx = jnp.pad(idx, ((0, _MP - _M), (0, 0)))
    idx4 = idx.T.reshape(_K, _NW, _NCH, _CH)

    vl = jnp.maximum(valid_lens, 1).astype(jnp.float32).reshape(_M, 1)
    ivl = jnp.pad(1.0 / vl, ((0, _MP - _M), (0, 0)), constant_values=1.0)

    h = _tc_embed(xp, W_embed)
    for l in range(Ws.shape[0]):
        agg = _sc_aggregate(h, idx4)
        h = _tc_layer(agg, h, ivl, Ws[l], Bs[l])
    probs = _tc_head(h, Wc1, bc1.reshape(1, _H), Wc2, bc2.reshape(1, _C))
    return probs[:_M].reshape(_B, _N, _C)


# speedup vs baseline: 42.0433x; 3.4932x over previous
import functools

import jax
import jax.numpy as jnp
from jax import lax
from jax.experimental import pallas as pl
from jax.experimental.pallas import tpu as pltpu
from jax.experimental.pallas import tpu_sc as plsc

_B, _N, _K, _D, _H, _C = 2, 10000, 16, 128, 128, 64
_M = _B * _N
_NW = 32
_MP = 20480
_HALF = _MP // 2
_CCH = 1024
_NCHK = _HALF // _CCH
_GRP = _CCH // 16
_W = _H // 2 // 16
_BN = 2048
_MASK = -65536


def _sc_aggregate_t(hTp, idxc):
    mesh = plsc.VectorSubcoreMesh(core_axis_name="c", subcore_axis_name="s")

    @functools.partial(
        pl.kernel,
        out_type=jax.ShapeDtypeStruct((_H, _MP), jnp.float32),
        mesh=mesh,
        scratch_types=[
            pltpu.VMEM((_W * _MP,), jnp.int32),
            pltpu.VMEM((_K, _CCH), jnp.int32),
            pltpu.VMEM((2 * _W, _CCH), jnp.float32),
        ],
        compiler_params=pltpu.CompilerParams(needs_layout_passes=False),
    )
    def agg_kernel(tab_hbm, idx_hbm, out_hbm, tab_v, idx_v, out_v):
        wid = lax.axis_index("s") * 2 + lax.axis_index("c")
        wr = (wid % 16) * _W
        half = wid // 16
        node0 = half * _HALF
        pltpu.sync_copy(tab_hbm.at[pl.ds(wr * _MP, _W * _MP)], tab_v)

        for chunk in range(_NCHK):
            pltpu.sync_copy(idx_hbm.at[half, chunk], idx_v)

            def group(g, carry):
                col = pl.ds(g * 16, 16)
                acc_lo = [None] * _W
                acc_hi = [None] * _W
                for kk in range(_K):
                    iv = idx_v[kk, col]
                    for w in range(_W):
                        word = plsc.load_gather(tab_v, [iv + w * _MP])
                        lo = plsc.bitcast(word << 16, jnp.float32)
                        hi = plsc.bitcast(word & _MASK, jnp.float32)
                        if kk == 0:
                            acc_lo[w], acc_hi[w] = lo, hi
                        else:
                            acc_lo[w] = acc_lo[w] + lo
                            acc_hi[w] = acc_hi[w] + hi
                for w in range(_W):
                    out_v[w, col] = acc_lo[w]
                    out_v[_W + w, col] = acc_hi[w]
                return carry

            lax.fori_loop(0, _GRP, group, 0)
            col0 = node0 + chunk * _CCH
            pltpu.sync_copy(
                out_v.at[pl.ds(0, _W)],
                out_hbm.at[pl.ds(wr, _W), pl.ds(col0, _CCH)])
            pltpu.sync_copy(
                out_v.at[pl.ds(_W, _W)],
                out_hbm.at[pl.ds(wr + 64, _W), pl.ds(col0, _CCH)])

    return agg_kernel(hTp, idxc)


def _pack_bf16_pairs(t):
    u = jax.lax.bitcast_convert_type(t, jnp.uint32)
    rne = lambda v: (v + jnp.uint32(0x7FFF) + ((v >> 16) & jnp.uint32(1))) >> 16
    word = rne(u[:64]) | (rne(u[64:]) << 16)
    return jax.lax.bitcast_convert_type(word, jnp.int32)


def _embed_body(x_ref, w_ref, ht_ref, pk_ref):
    x = x_ref[...]
    rs = jax.lax.dot_general(jnp.ones((1, _D), jnp.float32), x,
                             (((1,), (1,)), ((), ())),
                             preferred_element_type=jnp.float32)
    ht = jax.lax.dot_general(w_ref[...], x, (((0,), (1,)), ((), ())),
                             preferred_element_type=jnp.float32) / rs
    ht_ref[...] = ht
    pk_ref[...] = _pack_bf16_pairs(ht)


def _tc_embed_t(x, w):
    return pl.pallas_call(
        _embed_body,
        grid=(_MP // _BN,),
        in_specs=[
            pl.BlockSpec((_BN, _D), lambda i: (i, 0)),
            pl.BlockSpec((_D, _H), lambda i: (0, 0)),
        ],
        out_specs=[
            pl.BlockSpec((_H, _BN), lambda i: (0, i)),
            pl.BlockSpec((_H // 2, _BN), lambda i: (0, i)),
        ],
        out_shape=[
            jax.ShapeDtypeStruct((_H, _MP), jnp.float32),
            jax.ShapeDtypeStruct((_H // 2, _MP), jnp.int32),
        ],
    )(x, w)


def _layer_body(aggt_ref, ht_ref, ivl_ref, ws_ref, bs_ref, o_ref, pk_ref):
    a = aggt_ref[...] * ivl_ref[...]
    hn = jnp.maximum(
        jax.lax.dot_general(ws_ref[...], a, (((0,), (0,)), ((), ())),
                            preferred_element_type=jnp.float32)
        + jax.lax.dot_general(bs_ref[...], ht_ref[...],
                              (((0,), (0,)), ((), ())),
                              preferred_element_type=jnp.float32),
        0.0)
    o_ref[...] = hn
    pk_ref[...] = _pack_bf16_pairs(hn)


def _tc_layer_t(aggt, ht, ivl, ws, bs):
    return pl.pallas_call(
        _layer_body,
        grid=(_MP // _BN,),
        in_specs=[
            pl.BlockSpec((_H, _BN), lambda i: (0, i)),
            pl.BlockSpec((_H, _BN), lambda i: (0, i)),
            pl.BlockSpec((1, _BN), lambda i: (0, i)),
            pl.BlockSpec((_H, _H), lambda i: (0, 0)),
            pl.BlockSpec((_H, _H), lambda i: (0, 0)),
        ],
        out_specs=[
            pl.BlockSpec((_H, _BN), lambda i: (0, i)),
            pl.BlockSpec((_H // 2, _BN), lambda i: (0, i)),
        ],
        out_shape=[
            jax.ShapeDtypeStruct((_H, _MP), jnp.float32),
            jax.ShapeDtypeStruct((_H // 2, _MP), jnp.int32),
        ],
    )(aggt, ht, ivl, ws, bs)


def _head_body(ht_ref, w1_ref, b1_ref, w2_ref, b2_ref, o_ref):
    zt = jnp.maximum(
        jax.lax.dot_general(w1_ref[...], ht_ref[...],
                            (((0,), (0,)), ((), ())),
                            preferred_element_type=jnp.float32)
        + b1_ref[...],
        0.0)
    lg = jax.lax.dot_general(w2_ref[...], zt, (((0,), (0,)), ((), ())),
                             preferred_element_type=jnp.float32) + b2_ref[...]
    m = jnp.max(lg, axis=0, keepdims=True)
    e = jnp.exp(lg - m)
    o_ref[...] = e / jnp.sum(e, axis=0, keepdims=True)


def _tc_head_t(ht, w1, b1, w2, b2):
    return pl.pallas_call(
        _head_body,
        grid=(_MP // _BN,),
        in_specs=[
            pl.BlockSpec((_H, _BN), lambda i: (0, i)),
            pl.BlockSpec((_H, _H), lambda i: (0, 0)),
            pl.BlockSpec((_H, 1), lambda i: (0, 0)),
            pl.BlockSpec((_H, _C), lambda i: (0, 0)),
            pl.BlockSpec((_C, 1), lambda i: (0, 0)),
        ],
        out_specs=pl.BlockSpec((_C, _BN), lambda i: (0, i)),
        out_shape=jax.ShapeDtypeStruct((_C, _MP), jnp.float32),
    )(ht, w1, b1, w2, b2)


def kernel(vertex_feat, neighbors_idx, valid_lens, W_embed, Ws, Bs,
           Wc1, bc1, Wc2, bc2):
    x = vertex_feat.reshape(_M, _D)
    xp = jnp.pad(x, ((0, _MP - _M), (0, 0)), constant_values=1.0)

    offs = (jnp.arange(_B, dtype=jnp.int32) * _N)[:, None, None]
    idx = (neighbors_idx + offs).reshape(_M, _K)
    idx = jnp.pad(idx, ((0, _MP - _M), (0, 0)))
    idxc = idx.T.reshape(_K, 2, _NCHK, _CCH).transpose(1, 2, 0, 3)

    vl = jnp.maximum(valid_lens, 1).astype(jnp.float32).reshape(1, _M)
    ivl = jnp.pad(1.0 / vl, ((0, 0), (0, _MP - _M)), constant_values=1.0)

    ht, htp = _tc_embed_t(xp, W_embed)
    for l in range(Ws.shape[0]):
        aggt = _sc_aggregate_t(htp.reshape(-1), idxc)
        ht, htp = _tc_layer_t(aggt, ht, ivl, Ws[l], Bs[l])
    probst = _tc_head_t(ht, Wc1, bc1.reshape(_H, 1), Wc2, bc2.reshape(_C, 1))
    return probst[:, :_M].T.reshape(_B, _N, _C)
